# baseline (speedup 1.0000x reference)
import jax
import jax.numpy as jnp
from jax.experimental import pallas as pl
from jax.experimental.pallas import tpu as pltpu
from jax.experimental.pallas import tpu_sc as plsc

T = 2048
H = 2048
E = 8
K = 2
I = 768
TILE = 256
NPAD = T * K + E * TILE
NT = NPAD // TILE

_NEG_INF = float("-inf")


def _router_body(x_ref, gw_ref, slot0_ref, slot1_ref, w0_ref, w1_ref, cnt_ref):
    logits = jax.lax.dot_general(
        gw_ref[...], x_ref[...], (((1,), (1,)), ((), ())),
        preferred_element_type=jnp.float32)
    sub = jax.lax.broadcasted_iota(jnp.int32, (E, T), 0)
    m1 = jnp.max(logits, axis=0, keepdims=True)
    i1 = jnp.min(jnp.where(logits == m1, sub, E), axis=0, keepdims=True)
    masked = jnp.where(sub == i1, _NEG_INF, logits)
    m2 = jnp.max(masked, axis=0, keepdims=True)
    i2 = jnp.min(jnp.where(masked == m2, sub, E), axis=0, keepdims=True)
    w0 = 1.0 / (1.0 + jnp.exp(m2 - m1))
    w0_ref[...] = w0
    w1_ref[...] = 1.0 - w0
    m = ((sub == i1) | (sub == i2)).astype(jnp.int32)
    cx = m
    sh = 1
    while sh < T:
        z = jnp.zeros((E, sh), jnp.int32)
        cx = cx + jnp.concatenate([z, cx[:, :-sh]], axis=1)
        sh *= 2
    cx_excl = cx - m
    counts = cx[:, T - 1:T]
    ptiles = (counts + (TILE - 1)) // TILE
    padded = ptiles * TILE
    base = jnp.concatenate([jnp.zeros((1, 1), jnp.int32), padded[:-1]], axis=0)
    sh = 1
    while sh < E:
        z = jnp.zeros((sh, 1), jnp.int32)
        base = base + jnp.concatenate([z, base[:-sh]], axis=0)
        sh *= 2
    slotv = base + cx_excl
    slot0_ref[...] = jnp.sum(jnp.where(sub == i1, slotv, 0), axis=0,
                             keepdims=True)
    slot1_ref[...] = jnp.sum(jnp.where(sub == i2, slotv, 0), axis=0,
                             keepdims=True)
    cnt_ref[...] = counts


def _router(x, gate_w):
    return pl.pallas_call(
        _router_body,
        out_shape=(
            jax.ShapeDtypeStruct((1, T), jnp.int32),
            jax.ShapeDtypeStruct((1, T), jnp.int32),
            jax.ShapeDtypeStruct((1, T), jnp.float32),
            jax.ShapeDtypeStruct((1, T), jnp.float32),
            jax.ShapeDtypeStruct((E, 1), jnp.int32),
        ),
    )(x, gate_w)


def _vector_mesh():
    return plsc.VectorSubcoreMesh(core_axis_name="c", subcore_axis_name="s")


_NSUB = 32
_NIDX = T * K
_IPS = _NIDX // _NSUB


def _sc_scatter(x, slots_rs):
    w = 16
    n = _IPS // w

    @pl.kernel(out_type=jax.ShapeDtypeStruct((NPAD, H), jnp.float32),
               mesh=_vector_mesh(),
               scratch_types=[pltpu.VMEM((1, _IPS), jnp.int32),
                              pltpu.VMEM((w, H), jnp.float32),
                              pltpu.VMEM((w, H), jnp.float32),
                              pltpu.SemaphoreType.DMA,
                              pltpu.SemaphoreType.DMA,
                              pltpu.SemaphoreType.DMA,
                              pltpu.SemaphoreType.DMA])
    def scatter_kernel(x_hbm, i_hbm, o_hbm, idx_buf, buf_a, buf_b,
                       si_a, si_b, so_a, so_b):
        c = jax.lax.axis_index("c")
        s = jax.lax.axis_index("s")
        sub = c * 16 + s
        pltpu.sync_copy(i_hbm.at[pl.ds(sub, 1)], idx_buf)
        bufs, sis, sos = (buf_a, buf_b), (si_a, si_b), (so_a, so_b)

        def cp_in(j, b):
            return pltpu.make_async_copy(
                x_hbm.at[pl.ds(s * _IPS + j * w, w)], bufs[b], sis[b])

        def cp_out(j, b):
            return pltpu.make_async_copy(
                bufs[b], o_hbm.at[idx_buf.at[0, pl.ds(j * w, w)]], sos[b])

        cp_in(0, 0).start()
        for j in range(n):
            b = j % 2
            cp_in(j, b).wait()
            if j + 1 < n:
                if j >= 1:
                    cp_out(j - 1, 1 - b).wait()
                cp_in(j + 1, 1 - b).start()
            cp_out(j, b).start()
        cp_out(n - 2, n % 2).wait()
        cp_out(n - 1, (n - 1) % 2).wait()

    return scatter_kernel(x, slots_rs)


def _sc_gather(ys, slots_rs):
    w = 16
    n = _IPS // w

    @pl.kernel(out_type=jax.ShapeDtypeStruct((_NIDX, H), ys.dtype),
               mesh=_vector_mesh(),
               scratch_types=[pltpu.VMEM((1, _IPS), jnp.int32),
                              pltpu.VMEM((w, H), jnp.float32),
                              pltpu.VMEM((w, H), jnp.float32),
                              pltpu.SemaphoreType.DMA,
                              pltpu.SemaphoreType.DMA,
                              pltpu.SemaphoreType.DMA,
                              pltpu.SemaphoreType.DMA])
    def gather_kernel(y_hbm, i_hbm, o_hbm, idx_buf, buf_a, buf_b,
                      si_a, si_b, so_a, so_b):
        c = jax.lax.axis_index("c")
        s = jax.lax.axis_index("s")
        sub = c * 16 + s
        pltpu.sync_copy(i_hbm.at[pl.ds(sub, 1)], idx_buf)
        bufs, sis, sos = (buf_a, buf_b), (si_a, si_b), (so_a, so_b)

        def cp_in(j, b):
            return pltpu.make_async_copy(
                y_hbm.at[idx_buf.at[0, pl.ds(j * w, w)]], bufs[b], sis[b])

        def cp_out(j, b):
            return pltpu.make_async_copy(
                bufs[b], o_hbm.at[pl.ds(sub * _IPS + j * w, w)], sos[b])

        cp_in(0, 0).start()
        for j in range(n):
            b = j % 2
            cp_in(j, b).wait()
            if j + 1 < n:
                if j >= 1:
                    cp_out(j - 1, 1 - b).wait()
                cp_in(j + 1, 1 - b).start()
            cp_out(j, b).start()
        cp_out(n - 2, n % 2).wait()
        cp_out(n - 1, (n - 1) % 2).wait()

    return gather_kernel(ys, slots_rs)


def _ffn_body(te_ref, nu_ref, xs_ref, g_ref, u_ref, d_ref, ys_ref):
    g = pl.program_id(0)

    @pl.when(g < nu_ref[0])
    def _():
        xt = xs_ref[...].astype(jnp.bfloat16)
        gh = jax.lax.dot_general(xt, g_ref[0], (((1,), (1,)), ((), ())),
                                 preferred_element_type=jnp.float32)
        uh = jax.lax.dot_general(xt, u_ref[0], (((1,), (1,)), ((), ())),
                                 preferred_element_type=jnp.float32)
        act = (gh * jax.nn.sigmoid(gh) * uh).astype(jnp.bfloat16)
        ys_ref[...] = jax.lax.dot_general(
            act, d_ref[0], (((1,), (1,)), ((), ())),
            preferred_element_type=jnp.float32)


def _grouped_ffn(tile_expert, n_used, xs, gate_p, up_p, down_p):
    grid_spec = pltpu.PrefetchScalarGridSpec(
        num_scalar_prefetch=2,
        grid=(NT,),
        in_specs=[
            pl.BlockSpec((TILE, H), lambda g, te, nu: (g, 0)),
            pl.BlockSpec((1, I, H), lambda g, te, nu: (te[g], 0, 0)),
            pl.BlockSpec((1, I, H), lambda g, te, nu: (te[g], 0, 0)),
            pl.BlockSpec((1, H, I), lambda g, te, nu: (te[g], 0, 0)),
        ],
        out_specs=pl.BlockSpec((TILE, H), lambda g, te, nu: (g, 0)),
    )
    return pl.pallas_call(
        _ffn_body,
        grid_spec=grid_spec,
        out_shape=jax.ShapeDtypeStruct((NPAD, H), jnp.float32),
    )(tile_expert, n_used, xs, gate_p, up_p, down_p)


def _combine_body(y0_ref, y1_ref, w0_ref, w1_ref, o_ref):
    o_ref[...] = (w0_ref[...] * y0_ref[...].astype(jnp.float32)
                  + w1_ref[...] * y1_ref[...].astype(jnp.float32))


def _combine(yg, w0, w1):
    n = T // TILE
    return pl.pallas_call(
        _combine_body,
        grid=(n,),
        in_specs=[
            pl.BlockSpec((TILE, H), lambda g: (g, 0)),
            pl.BlockSpec((TILE, H), lambda g: (g + n, 0)),
            pl.BlockSpec((TILE, 1), lambda g: (g, 0)),
            pl.BlockSpec((TILE, 1), lambda g: (g, 0)),
        ],
        out_specs=pl.BlockSpec((TILE, H), lambda g: (g, 0)),
        out_shape=jax.ShapeDtypeStruct((T, H), jnp.float32),
    )(yg, yg, w0, w1)


def kernel(hidden_states, gate_w, gate_proj, up_proj, down_proj):
    b, s, h = hidden_states.shape
    x = hidden_states.reshape(T, H)

    slot0, slot1, w0, w1, counts = _router(x, gate_w)

    counts = counts.reshape(E)
    ptiles = (counts + (TILE - 1)) // TILE
    tile_end = jnp.cumsum(ptiles)
    gidx = jnp.arange(NT, dtype=jnp.int32)
    tile_expert = jnp.minimum(
        jnp.sum((gidx[:, None] >= tile_end[None, :]).astype(jnp.int32), axis=1),
        E - 1).astype(jnp.int32)

    slots = jnp.concatenate([slot0, slot1], axis=1).reshape(_NSUB, _IPS)
    xs = _sc_scatter(x, slots)
    n_used = tile_end[E - 1].reshape(1).astype(jnp.int32)
    ys = _grouped_ffn(tile_expert, n_used, xs,
                      gate_proj.astype(jnp.bfloat16),
                      up_proj.astype(jnp.bfloat16),
                      down_proj.astype(jnp.bfloat16))
    yg = _sc_gather(ys, slots)
    out = _combine(yg, w0.reshape(T, 1), w1.reshape(T, 1))
    return out.reshape(b, s, h)


# speedup vs baseline: 1.1727x; 1.1727x over previous
import jax
import jax.numpy as jnp
from jax.experimental import pallas as pl
from jax.experimental.pallas import tpu as pltpu
from jax.experimental.pallas import tpu_sc as plsc

T = 2048
H = 2048
E = 8
K = 2
I = 768
TILE = 256
NPAD = T * K + E * TILE
NT = NPAD // TILE

_NEG_INF = float("-inf")


def _router_body(x_ref, gw_ref, slot0_ref, slot1_ref, w0_ref, w1_ref, cnt_ref):
    logits = jax.lax.dot_general(
        gw_ref[...], x_ref[...], (((1,), (1,)), ((), ())),
        preferred_element_type=jnp.float32)
    sub = jax.lax.broadcasted_iota(jnp.int32, (E, T), 0)
    m1 = jnp.max(logits, axis=0, keepdims=True)
    i1 = jnp.min(jnp.where(logits == m1, sub, E), axis=0, keepdims=True)
    masked = jnp.where(sub == i1, _NEG_INF, logits)
    m2 = jnp.max(masked, axis=0, keepdims=True)
    i2 = jnp.min(jnp.where(masked == m2, sub, E), axis=0, keepdims=True)
    w0 = 1.0 / (1.0 + jnp.exp(m2 - m1))
    w0_ref[...] = w0
    w1_ref[...] = 1.0 - w0
    m = ((sub == i1) | (sub == i2)).astype(jnp.int32)
    cx = m
    sh = 1
    while sh < T:
        z = jnp.zeros((E, sh), jnp.int32)
        cx = cx + jnp.concatenate([z, cx[:, :-sh]], axis=1)
        sh *= 2
    cx_excl = cx - m
    counts = cx[:, T - 1:T]
    ptiles = (counts + (TILE - 1)) // TILE
    padded = ptiles * TILE
    base = jnp.concatenate([jnp.zeros((1, 1), jnp.int32), padded[:-1]], axis=0)
    sh = 1
    while sh < E:
        z = jnp.zeros((sh, 1), jnp.int32)
        base = base + jnp.concatenate([z, base[:-sh]], axis=0)
        sh *= 2
    slotv = base + cx_excl
    slot0_ref[...] = jnp.sum(jnp.where(sub == i1, slotv, 0), axis=0,
                             keepdims=True)
    slot1_ref[...] = jnp.sum(jnp.where(sub == i2, slotv, 0), axis=0,
                             keepdims=True)
    cnt_ref[...] = counts


def _router(x, gate_w):
    return pl.pallas_call(
        _router_body,
        out_shape=(
            jax.ShapeDtypeStruct((1, T), jnp.int32),
            jax.ShapeDtypeStruct((1, T), jnp.int32),
            jax.ShapeDtypeStruct((1, T), jnp.float32),
            jax.ShapeDtypeStruct((1, T), jnp.float32),
            jax.ShapeDtypeStruct((E, 1), jnp.int32),
        ),
    )(x, gate_w)


def _vector_mesh():
    return plsc.VectorSubcoreMesh(core_axis_name="c", subcore_axis_name="s")


_NSUB = 32
_NIDX = T * K
_IPS = _NIDX // _NSUB


def _sc_scatter(x, slots_rs):
    w = 16
    n = _IPS // w

    @pl.kernel(out_type=jax.ShapeDtypeStruct((NPAD, H), jnp.float32),
               mesh=_vector_mesh(),
               scratch_types=[pltpu.VMEM((1, _IPS), jnp.int32),
                              pltpu.VMEM((w, H), jnp.float32),
                              pltpu.VMEM((w, H), jnp.float32),
                              pltpu.SemaphoreType.DMA,
                              pltpu.SemaphoreType.DMA,
                              pltpu.SemaphoreType.DMA,
                              pltpu.SemaphoreType.DMA])
    def scatter_kernel(x_hbm, i_hbm, o_hbm, idx_buf, buf_a, buf_b,
                       si_a, si_b, so_a, so_b):
        c = jax.lax.axis_index("c")
        s = jax.lax.axis_index("s")
        sub = c * 16 + s
        pltpu.sync_copy(i_hbm.at[pl.ds(sub, 1)], idx_buf)
        bufs, sis, sos = (buf_a, buf_b), (si_a, si_b), (so_a, so_b)

        def cp_in(j, b):
            return pltpu.make_async_copy(
                x_hbm.at[pl.ds(s * _IPS + j * w, w)], bufs[b], sis[b])

        def cp_out(j, b):
            return pltpu.make_async_copy(
                bufs[b], o_hbm.at[idx_buf.at[0, pl.ds(j * w, w)]], sos[b])

        cp_in(0, 0).start()
        for j in range(n):
            b = j % 2
            cp_in(j, b).wait()
            if j + 1 < n:
                if j >= 1:
                    cp_out(j - 1, 1 - b).wait()
                cp_in(j + 1, 1 - b).start()
            cp_out(j, b).start()
        cp_out(n - 2, n % 2).wait()
        cp_out(n - 1, (n - 1) % 2).wait()

    return scatter_kernel(x, slots_rs)


def _sc_gather(ys, slots_rs):
    w = 16
    n = _IPS // w

    @pl.kernel(out_type=jax.ShapeDtypeStruct((_NIDX, H), ys.dtype),
               mesh=_vector_mesh(),
               scratch_types=[pltpu.VMEM((1, _IPS), jnp.int32),
                              pltpu.VMEM((w, H), jnp.float32),
                              pltpu.VMEM((w, H), jnp.float32),
                              pltpu.SemaphoreType.DMA,
                              pltpu.SemaphoreType.DMA,
                              pltpu.SemaphoreType.DMA,
                              pltpu.SemaphoreType.DMA])
    def gather_kernel(y_hbm, i_hbm, o_hbm, idx_buf, buf_a, buf_b,
                      si_a, si_b, so_a, so_b):
        c = jax.lax.axis_index("c")
        s = jax.lax.axis_index("s")
        sub = c * 16 + s
        pltpu.sync_copy(i_hbm.at[pl.ds(sub, 1)], idx_buf)
        bufs, sis, sos = (buf_a, buf_b), (si_a, si_b), (so_a, so_b)

        def cp_in(j, b):
            return pltpu.make_async_copy(
                y_hbm.at[idx_buf.at[0, pl.ds(j * w, w)]], bufs[b], sis[b])

        def cp_out(j, b):
            return pltpu.make_async_copy(
                bufs[b], o_hbm.at[pl.ds(sub * _IPS + j * w, w)], sos[b])

        cp_in(0, 0).start()
        for j in range(n):
            b = j % 2
            cp_in(j, b).wait()
            if j + 1 < n:
                if j >= 1:
                    cp_out(j - 1, 1 - b).wait()
                cp_in(j + 1, 1 - b).start()
            cp_out(j, b).start()
        cp_out(n - 2, n % 2).wait()
        cp_out(n - 1, (n - 1) % 2).wait()

    return gather_kernel(ys, slots_rs)


def _ffn_body(te_ref, nu_ref, xs_ref, g_ref, u_ref, d_ref, ys_ref,
              gb_ref, db_ref):
    g = pl.program_id(0)

    @pl.when(g < nu_ref[0])
    def _():
        changed = jnp.logical_or(
            g == 0, te_ref[g] != te_ref[jnp.maximum(g - 1, 0)])

        @pl.when(changed)
        def _():
            gb_ref[0:I, :] = g_ref[0].astype(jnp.bfloat16)
            gb_ref[I:2 * I, :] = u_ref[0].astype(jnp.bfloat16)
            db_ref[...] = d_ref[0].astype(jnp.bfloat16)

        xt = xs_ref[...].astype(jnp.bfloat16)
        gu = jax.lax.dot_general(xt, gb_ref[...], (((1,), (1,)), ((), ())),
                                 preferred_element_type=jnp.float32)
        gh = gu[:, 0:I]
        uh = gu[:, I:2 * I]
        act = (gh * jax.nn.sigmoid(gh) * uh).astype(jnp.bfloat16)
        ys_ref[...] = jax.lax.dot_general(
            act, db_ref[...], (((1,), (1,)), ((), ())),
            preferred_element_type=jnp.float32)


def _grouped_ffn(tile_expert, n_used, xs, gate_p, up_p, down_p):
    grid_spec = pltpu.PrefetchScalarGridSpec(
        num_scalar_prefetch=2,
        grid=(NT,),
        in_specs=[
            pl.BlockSpec((TILE, H), lambda g, te, nu: (g, 0)),
            pl.BlockSpec((1, I, H), lambda g, te, nu: (te[g], 0, 0)),
            pl.BlockSpec((1, I, H), lambda g, te, nu: (te[g], 0, 0)),
            pl.BlockSpec((1, H, I), lambda g, te, nu: (te[g], 0, 0)),
        ],
        out_specs=pl.BlockSpec((TILE, H), lambda g, te, nu: (g, 0)),
        scratch_shapes=[
            pltpu.VMEM((2 * I, H), jnp.bfloat16),
            pltpu.VMEM((H, I), jnp.bfloat16),
        ],
    )
    return pl.pallas_call(
        _ffn_body,
        grid_spec=grid_spec,
        out_shape=jax.ShapeDtypeStruct((NPAD, H), jnp.float32),
    )(tile_expert, n_used, xs, gate_p, up_p, down_p)


def _combine_body(y0_ref, y1_ref, w0_ref, w1_ref, o_ref):
    o_ref[...] = (w0_ref[...] * y0_ref[...].astype(jnp.float32)
                  + w1_ref[...] * y1_ref[...].astype(jnp.float32))


def _combine(yg, w0, w1):
    n = T // TILE
    return pl.pallas_call(
        _combine_body,
        grid=(n,),
        in_specs=[
            pl.BlockSpec((TILE, H), lambda g: (g, 0)),
            pl.BlockSpec((TILE, H), lambda g: (g + n, 0)),
            pl.BlockSpec((TILE, 1), lambda g: (g, 0)),
            pl.BlockSpec((TILE, 1), lambda g: (g, 0)),
        ],
        out_specs=pl.BlockSpec((TILE, H), lambda g: (g, 0)),
        out_shape=jax.ShapeDtypeStruct((T, H), jnp.float32),
    )(yg, yg, w0, w1)


def kernel(hidden_states, gate_w, gate_proj, up_proj, down_proj):
    b, s, h = hidden_states.shape
    x = hidden_states.reshape(T, H)

    slot0, slot1, w0, w1, counts = _router(x, gate_w)

    counts = counts.reshape(E)
    ptiles = (counts + (TILE - 1)) // TILE
    tile_end = jnp.cumsum(ptiles)
    gidx = jnp.arange(NT, dtype=jnp.int32)
    tile_expert = jnp.minimum(
        jnp.sum((gidx[:, None] >= tile_end[None, :]).astype(jnp.int32), axis=1),
        E - 1).astype(jnp.int32)

    slots = jnp.concatenate([slot0, slot1], axis=1).reshape(_NSUB, _IPS)
    xs = _sc_scatter(x, slots)
    n_used = tile_end[E - 1].reshape(1).astype(jnp.int32)
    ys = _grouped_ffn(tile_expert, n_used, xs, gate_proj, up_proj, down_proj)
    yg = _sc_gather(ys, slots)
    out = _combine(yg, w0.reshape(T, 1), w1.reshape(T, 1))
    return out.reshape(b, s, h)
